# baseline (speedup 1.0000x reference)
import functools

import jax
import jax.numpy as jnp
from jax import lax
from jax.experimental import pallas as pl
from jax.experimental.pallas import tpu as pltpu
from jax.experimental.pallas import tpu_sc as plsc

D_MODEL = 64
D_WIDE = 128
SCALE = 8.0
NUM_CORES = 2
NUM_SUBCORES = 16
LANES = 16
NUM_WORKERS = NUM_CORES * NUM_SUBCORES
TOK = 128


def _embed_lookup(idx, table_wide, n_rows, n_cols):
    n_tok = TOK * n_cols
    assert n_rows == NUM_WORKERS * TOK and n_cols % 2 == 0

    mesh = plsc.VectorSubcoreMesh(core_axis_name="c", subcore_axis_name="s")

    @functools.partial(
        pl.kernel,
        mesh=mesh,
        compiler_params=pltpu.CompilerParams(needs_layout_passes=False),
        out_type=jax.ShapeDtypeStruct(
            (n_cols, 8, NUM_WORKERS, 8, TOK), jnp.float32
        ),
        scratch_types=[
            pltpu.VMEM((n_tok,), jnp.int32),
            pltpu.VMEM((n_tok,), jnp.int32),
            pltpu.VMEM((TOK, D_WIDE), jnp.float32),
            pltpu.VMEM((TOK, D_WIDE), jnp.float32),
            pltpu.VMEM((8, 8, TOK), jnp.float32),
            pltpu.VMEM((8, 8, TOK), jnp.float32),
        ]
        + [pltpu.SemaphoreType.DMA] * 5,
    )
    def k(idx_hbm, table_hbm, out_hbm, idx_v, idxt_v, w0, w1, t0, t1, *sems):
        wbufs = (w0, w1)
        tbufs = (t0, t1)
        gsem = sems[0:2]
        ssem = sems[2:4]
        isem = sems[4]

        wid = lax.axis_index("s") * NUM_CORES + lax.axis_index("c")
        tok0 = wid * n_tok
        pltpu.make_async_copy(
            idx_hbm.at[pl.ds(tok0, n_tok)], idx_v, isem
        ).start()
        pltpu.make_async_copy(
            idx_hbm.at[pl.ds(tok0, n_tok)], idx_v, isem
        ).wait()

        iota = lax.iota(jnp.int32, LANES)

        for jg in range(TOK // LANES):
            base = (jg * LANES + iota) * n_cols

            @pl.loop(0, n_cols)
            def _(s):
                v = plsc.load_gather(idx_v, [base + s])
                idxt_v.at[pl.ds(s * TOK + jg * LANES, LANES)][...] = v

        def gather_src(s):
            return table_hbm.at[idxt_v.at[pl.ds(s * TOK, TOK)]]

        def out_dst(s):
            return out_hbm.at[s, :, wid]

        for b in range(2):
            pltpu.async_copy(gather_src(b), wbufs[b], gsem[b])

        @pl.loop(0, n_cols, step=2)
        def _(s):
            for b in range(2):
                sb = s + b
                pltpu.make_async_copy(gather_src(sb), wbufs[b], gsem[b]).wait()

                @pl.when(sb >= 2)
                def _():
                    pltpu.make_async_copy(
                        tbufs[b], out_dst(sb - 2), ssem[b]
                    ).wait()

                for jg in range(TOK // LANES):
                    tokv = jg * LANES + iota

                    @pl.loop(0, 8)
                    def _(dr):
                        for dg in range(8):
                            col = dg * 8 + dr + iota * 0
                            v = plsc.load_gather(wbufs[b], [tokv, col])
                            tbufs[b].at[dg, dr, pl.ds(jg * LANES, LANES)][
                                ...
                            ] = v * SCALE

                pltpu.async_copy(tbufs[b], out_dst(sb), ssem[b])

                @pl.when(sb + 2 < n_cols)
                def _():
                    pltpu.async_copy(gather_src(sb + 2), wbufs[b], gsem[b])

        for b in range(2):
            pltpu.make_async_copy(
                tbufs[b], out_dst(n_cols - 2 + b), ssem[b]
            ).wait()

    return k(idx, table_wide)


def kernel(x, table):
    n_rows, n_cols = x.shape
    table_wide = jnp.pad(table, ((0, 0), (0, D_WIDE - D_MODEL)))
    idx = x.reshape(n_rows * n_cols).astype(jnp.int32)
    out5 = _embed_lookup(idx, table_wide, n_rows, n_cols)
    return out5.transpose(2, 4, 0, 1, 3).reshape(n_rows, n_cols, D_MODEL)


# speedup vs baseline: 1.1317x; 1.1317x over previous
import functools

import jax
import jax.numpy as jnp
from jax import lax
from jax.experimental import pallas as pl
from jax.experimental.pallas import tpu as pltpu
from jax.experimental.pallas import tpu_sc as plsc

D_MODEL = 64
D_WIDE = 128
SCALE = 8.0
NUM_CORES = 2
NUM_SUBCORES = 16
LANES = 16
NUM_WORKERS = NUM_CORES * NUM_SUBCORES
TOK = 128


def _embed_lookup(idx, table_wide, n_rows, n_cols):
    n_tok = TOK * n_cols
    assert n_rows == NUM_WORKERS * TOK and n_cols % 2 == 0

    mesh = plsc.VectorSubcoreMesh(core_axis_name="c", subcore_axis_name="s")

    @functools.partial(
        pl.kernel,
        mesh=mesh,
        compiler_params=pltpu.CompilerParams(needs_layout_passes=False),
        out_type=jax.ShapeDtypeStruct(
            (n_cols, 8, NUM_WORKERS, 8, TOK), jnp.float32
        ),
        scratch_types=[
            pltpu.VMEM((n_tok,), jnp.int32),
            pltpu.VMEM((n_tok,), jnp.int32),
            pltpu.VMEM((TOK, D_WIDE), jnp.float32),
            pltpu.VMEM((TOK, D_WIDE), jnp.float32),
            pltpu.VMEM((8, 8, TOK + 8), jnp.float32),
            pltpu.VMEM((8, 8, TOK + 8), jnp.float32),
        ]
        + [pltpu.SemaphoreType.DMA] * 5,
    )
    def k(idx_hbm, table_hbm, out_hbm, idx_v, idxt_v, w0, w1, t0, t1, *sems):
        wbufs = (w0, w1)
        tbufs = (t0, t1)
        gsem = sems[0:2]
        ssem = sems[2:4]
        isem = sems[4]

        wid = lax.axis_index("s") * NUM_CORES + lax.axis_index("c")
        tok0 = wid * n_tok
        pltpu.make_async_copy(
            idx_hbm.at[pl.ds(tok0, n_tok)], idx_v, isem
        ).start()
        pltpu.make_async_copy(
            idx_hbm.at[pl.ds(tok0, n_tok)], idx_v, isem
        ).wait()

        iota = lax.iota(jnp.int32, LANES)

        for jg in range(TOK // LANES):
            base = (jg * LANES + iota) * n_cols

            @pl.loop(0, n_cols)
            def _(s):
                v = plsc.load_gather(idx_v, [base + s])
                idxt_v.at[pl.ds(s * TOK + jg * LANES, LANES)][...] = v

        def gather_src(s):
            return table_hbm.at[idxt_v.at[pl.ds(s * TOK, TOK)]]

        def out_dst(s):
            return out_hbm.at[s, :, wid]

        for b in range(2):
            pltpu.async_copy(gather_src(b), wbufs[b], gsem[b])

        @pl.loop(0, n_cols, step=2)
        def _(s):
            for b in range(2):
                sb = s + b
                pltpu.make_async_copy(gather_src(sb), wbufs[b], gsem[b]).wait()

                @pl.when(sb >= 2)
                def _():
                    pltpu.make_async_copy(
                        tbufs[b].at[:, :, pl.ds(0, TOK)],
                        out_dst(sb - 2),
                        ssem[b],
                    ).wait()

                @pl.loop(0, TOK)
                def _(j):
                    jv = j + iota * 0
                    for c in range(D_MODEL // LANES):
                        dv = c * LANES + iota
                        v = wbufs[b].at[j, pl.ds(c * LANES, LANES)][...]
                        plsc.store_scatter(
                            tbufs[b], [dv >> 3, dv & 7, jv], v * SCALE
                        )

                pltpu.async_copy(
                    tbufs[b].at[:, :, pl.ds(0, TOK)], out_dst(sb), ssem[b]
                )

                @pl.when(sb + 2 < n_cols)
                def _():
                    pltpu.async_copy(gather_src(sb + 2), wbufs[b], gsem[b])

        for b in range(2):
            pltpu.make_async_copy(
                tbufs[b].at[:, :, pl.ds(0, TOK)],
                out_dst(n_cols - 2 + b),
                ssem[b],
            ).wait()

    return k(idx, table_wide)


def kernel(x, table):
    n_rows, n_cols = x.shape
    table_wide = jnp.pad(table, ((0, 0), (0, D_WIDE - D_MODEL)))
    idx = x.reshape(n_rows * n_cols).astype(jnp.int32)
    out5 = _embed_lookup(idx, table_wide, n_rows, n_cols)
    return out5.transpose(2, 4, 0, 1, 3).reshape(n_rows, n_cols, D_MODEL)


# speedup vs baseline: 1.1448x; 1.0116x over previous
import functools

import jax
import jax.numpy as jnp
from jax import lax
from jax.experimental import pallas as pl
from jax.experimental.pallas import tpu as pltpu
from jax.experimental.pallas import tpu_sc as plsc

D_MODEL = 64
D_WIDE = 128
SCALE = 8.0
NUM_CORES = 2
NUM_SUBCORES = 16
LANES = 16
NUM_WORKERS = NUM_CORES * NUM_SUBCORES
TOK = 128


def _embed_lookup(idx, table_wide, n_rows, n_cols):
    n_tok = TOK * n_cols
    assert n_rows == NUM_WORKERS * TOK and n_cols % 2 == 0

    mesh = plsc.VectorSubcoreMesh(core_axis_name="c", subcore_axis_name="s")

    @functools.partial(
        pl.kernel,
        mesh=mesh,
        compiler_params=pltpu.CompilerParams(needs_layout_passes=False),
        out_type=jax.ShapeDtypeStruct(
            (n_cols, 8, NUM_WORKERS, 8, TOK), jnp.float32
        ),
        scratch_types=[
            pltpu.VMEM((n_tok,), jnp.int32),
            pltpu.VMEM((n_tok,), jnp.int32),
            pltpu.VMEM((TOK, D_WIDE), jnp.float32),
            pltpu.VMEM((TOK, D_WIDE), jnp.float32),
            pltpu.VMEM((8, 8, TOK + 8), jnp.float32),
            pltpu.VMEM((8, 8, TOK + 8), jnp.float32),
        ]
        + [pltpu.SemaphoreType.DMA] * 5,
    )
    def k(idx_hbm, table_hbm, out_hbm, idx_v, idxt_v, w0, w1, t0, t1, *sems):
        wbufs = (w0, w1)
        tbufs = (t0, t1)
        gsem = sems[0:2]
        ssem = sems[2:4]
        isem = sems[4]

        wid = lax.axis_index("s") * NUM_CORES + lax.axis_index("c")
        tok0 = wid * n_tok
        pltpu.make_async_copy(
            idx_hbm.at[pl.ds(tok0, n_tok)], idx_v, isem
        ).start()
        pltpu.make_async_copy(
            idx_hbm.at[pl.ds(tok0, n_tok)], idx_v, isem
        ).wait()

        iota = lax.iota(jnp.int32, LANES)

        for jg in range(TOK // LANES):
            base = (jg * LANES + iota) * n_cols

            @pl.loop(0, n_cols)
            def _(s):
                v = plsc.load_gather(idx_v, [base + s])
                idxt_v.at[pl.ds(s * TOK + jg * LANES, LANES)][...] = v

        def gather_src(s):
            return table_hbm.at[idxt_v.at[pl.ds(s * TOK, TOK)]]

        def out_dst(s):
            return out_hbm.at[s, :, wid]

        for b in range(2):
            pltpu.async_copy(gather_src(b), wbufs[b], gsem[b])

        @pl.loop(0, n_cols, step=2)
        def _(s):
            for b in range(2):
                sb = s + b
                pltpu.make_async_copy(gather_src(sb), wbufs[b], gsem[b]).wait()

                @pl.when(sb >= 2)
                def _():
                    pltpu.make_async_copy(
                        tbufs[b].at[:, :, pl.ds(0, TOK)],
                        out_dst(sb - 2),
                        ssem[b],
                    ).wait()

                @pl.loop(0, TOK, step=4)
                def _(j):
                    for u in range(4):
                        jv = (j + u) + iota * 0
                        for c in range(D_MODEL // LANES):
                            dv = c * LANES + iota
                            v = wbufs[b].at[j + u, pl.ds(c * LANES, LANES)][
                                ...
                            ]
                            plsc.store_scatter(
                                tbufs[b], [dv >> 3, dv & 7, jv], v * SCALE
                            )

                pltpu.async_copy(
                    tbufs[b].at[:, :, pl.ds(0, TOK)], out_dst(sb), ssem[b]
                )

                @pl.when(sb + 2 < n_cols)
                def _():
                    pltpu.async_copy(gather_src(sb + 2), wbufs[b], gsem[b])

        for b in range(2):
            pltpu.make_async_copy(
                tbufs[b].at[:, :, pl.ds(0, TOK)],
                out_dst(n_cols - 2 + b),
                ssem[b],
            ).wait()

    return k(idx, table_wide)


def kernel(x, table):
    n_rows, n_cols = x.shape
    table_wide = jnp.pad(table, ((0, 0), (0, D_WIDE - D_MODEL)))
    idx = x.reshape(n_rows * n_cols).astype(jnp.int32)
    out5 = _embed_lookup(idx, table_wide, n_rows, n_cols)
    return out5.transpose(2, 4, 0, 1, 3).reshape(n_rows, n_cols, D_MODEL)


# speedup vs baseline: 1.8704x; 1.6338x over previous
import functools

import jax
import jax.numpy as jnp
from jax import lax
from jax.experimental import pallas as pl
from jax.experimental.pallas import tpu as pltpu
from jax.experimental.pallas import tpu_sc as plsc

D_MODEL = 64
D_WIDE = 128
SCALE = 8.0
NUM_CORES = 2
NUM_SUBCORES = 16
LANES = 16
NUM_WORKERS = NUM_CORES * NUM_SUBCORES
NBUF = 2


def _embed_lookup(idx, table_wide, n_rows, n_cols):
    rows_per_w = n_rows // NUM_WORKERS
    n_tok = rows_per_w * n_cols
    assert n_rows % NUM_WORKERS == 0 and rows_per_w % NBUF == 0

    mesh = plsc.VectorSubcoreMesh(core_axis_name="c", subcore_axis_name="s")

    @functools.partial(
        pl.kernel,
        mesh=mesh,
        out_type=jax.ShapeDtypeStruct((n_rows, n_cols, D_WIDE), jnp.float32),
        scratch_types=[
            pltpu.VMEM((n_tok,), jnp.int32),
        ]
        + [pltpu.VMEM((n_cols, D_WIDE), jnp.float32)] * NBUF
        + [pltpu.SemaphoreType.DMA] * (2 * NBUF + 1),
    )
    def k(idx_hbm, table_hbm, out_hbm, idx_v, *bufs_and_sems):
        bufs = bufs_and_sems[:NBUF]
        gsem = bufs_and_sems[NBUF : 2 * NBUF]
        ssem = bufs_and_sems[2 * NBUF : 3 * NBUF]
        isem = bufs_and_sems[3 * NBUF]

        wid = lax.axis_index("s") * NUM_CORES + lax.axis_index("c")
        row0 = wid * rows_per_w
        pltpu.make_async_copy(
            idx_hbm.at[pl.ds(row0 * n_cols, n_tok)], idx_v, isem
        ).start()
        pltpu.make_async_copy(
            idx_hbm.at[pl.ds(row0 * n_cols, n_tok)], idx_v, isem
        ).wait()

        def gather_src(r):
            return table_hbm.at[idx_v.at[pl.ds(r * n_cols, n_cols)]]

        def out_dst(r):
            return out_hbm.at[row0 + r]

        for b in range(NBUF):
            pltpu.async_copy(gather_src(b), bufs[b], gsem[b])

        @pl.loop(0, rows_per_w, step=NBUF)
        def _(w):
            for b in range(NBUF):
                wb = w + b
                pltpu.make_async_copy(gather_src(wb), bufs[b], gsem[b]).wait()

                @pl.loop(0, n_cols)
                def _(r):
                    for c in range(0, D_MODEL, LANES):
                        slc = (pl.ds(r, 1), pl.ds(c, LANES))
                        bufs[b].at[slc][...] = bufs[b].at[slc][...] * SCALE

                pltpu.async_copy(bufs[b], out_dst(wb), ssem[b])

            for b in range(NBUF):
                wb = w + b
                pltpu.make_async_copy(bufs[b], out_dst(wb), ssem[b]).wait()

                @pl.when(wb + NBUF < rows_per_w)
                def _():
                    pltpu.async_copy(gather_src(wb + NBUF), bufs[b], gsem[b])

    return k(idx, table_wide)


def kernel(x, table):
    n_rows, n_cols = x.shape
    table_wide = jnp.pad(table, ((0, 0), (0, D_WIDE - D_MODEL)))
    idx = x.reshape(n_rows * n_cols).astype(jnp.int32)
    out_wide = _embed_lookup(idx, table_wide, n_rows, n_cols)
    return out_wide[:, :, :D_MODEL]


# speedup vs baseline: 1.8801x; 1.0052x over previous
import functools

import jax
import jax.numpy as jnp
from jax import lax
from jax.experimental import pallas as pl
from jax.experimental.pallas import tpu as pltpu
from jax.experimental.pallas import tpu_sc as plsc

D_MODEL = 64
D_WIDE = 128
SCALE = 8.0
NUM_CORES = 2
NUM_SUBCORES = 16
LANES = 16
NUM_WORKERS = NUM_CORES * NUM_SUBCORES
NBUF = 4


def _embed_lookup(idx, table_wide, n_rows, n_cols):
    rows_per_w = n_rows // NUM_WORKERS
    n_tok = rows_per_w * n_cols
    assert n_rows % NUM_WORKERS == 0 and rows_per_w % NBUF == 0

    mesh = plsc.VectorSubcoreMesh(core_axis_name="c", subcore_axis_name="s")

    @functools.partial(
        pl.kernel,
        mesh=mesh,
        out_type=jax.ShapeDtypeStruct((n_rows, n_cols, D_WIDE), jnp.float32),
        scratch_types=[
            pltpu.VMEM((n_tok,), jnp.int32),
        ]
        + [pltpu.VMEM((n_cols, D_WIDE), jnp.float32)] * NBUF
        + [pltpu.SemaphoreType.DMA] * (2 * NBUF + 1),
    )
    def k(idx_hbm, table_hbm, out_hbm, idx_v, *bufs_and_sems):
        bufs = bufs_and_sems[:NBUF]
        gsem = bufs_and_sems[NBUF : 2 * NBUF]
        ssem = bufs_and_sems[2 * NBUF : 3 * NBUF]
        isem = bufs_and_sems[3 * NBUF]

        wid = lax.axis_index("s") * NUM_CORES + lax.axis_index("c")
        row0 = wid * rows_per_w
        pltpu.make_async_copy(
            idx_hbm.at[pl.ds(row0 * n_cols, n_tok)], idx_v, isem
        ).start()
        pltpu.make_async_copy(
            idx_hbm.at[pl.ds(row0 * n_cols, n_tok)], idx_v, isem
        ).wait()

        def gather_src(r):
            return table_hbm.at[idx_v.at[pl.ds(r * n_cols, n_cols)]]

        def out_dst(r):
            return out_hbm.at[row0 + r]

        for b in range(NBUF):
            pltpu.async_copy(gather_src(b), bufs[b], gsem[b])

        @pl.loop(0, rows_per_w, step=NBUF)
        def _(w):
            for b in range(NBUF):
                wb = w + b
                pltpu.make_async_copy(gather_src(wb), bufs[b], gsem[b]).wait()

                @pl.loop(0, n_cols)
                def _(r):
                    for c in range(0, D_MODEL, LANES):
                        slc = (pl.ds(r, 1), pl.ds(c, LANES))
                        bufs[b].at[slc][...] = bufs[b].at[slc][...] * SCALE

                pltpu.async_copy(bufs[b], out_dst(wb), ssem[b])

            for b in range(NBUF):
                wb = w + b
                pltpu.make_async_copy(bufs[b], out_dst(wb), ssem[b]).wait()

                @pl.when(wb + NBUF < rows_per_w)
                def _():
                    pltpu.async_copy(gather_src(wb + NBUF), bufs[b], gsem[b])

    return k(idx, table_wide)


def kernel(x, table):
    n_rows, n_cols = x.shape
    table_wide = jnp.pad(table, ((0, 0), (0, D_WIDE - D_MODEL)))
    idx = x.reshape(n_rows * n_cols).astype(jnp.int32)
    out_wide = _embed_lookup(idx, table_wide, n_rows, n_cols)
    return out_wide[:, :, :D_MODEL]
